# Initial kernel scaffold; baseline (speedup 1.0000x reference)
import jax
import jax.numpy as jnp
from jax.experimental import pallas as pl


def kernel(z, embedding_weight):
    raise NotImplementedError("write your pallas kernel here")


# speedup vs baseline: 1.6015x; 1.6015x over previous
import functools

import jax
import jax.numpy as jnp
from jax import lax
from jax.experimental import pallas as pl
from jax.experimental.pallas import tpu as pltpu
from jax.experimental.pallas import tpu_sc as plsc

NUM_CODES = 8192
DIM = 32
ROW_BLK = 256
NUM_ROWS = 8192


def _vq_block(z_ref, z2_ref, et_ref, enc_ref, idx_ref, e2_ref):
    i = pl.program_id(0)

    @pl.when(i == 0)
    def _():
        et = et_ref[...]
        e2_ref[...] = jnp.sum(et * et, axis=0, keepdims=True)

    zb = z_ref[...]
    dot = jnp.dot(zb, et_ref[...],
                  preferred_element_type=jnp.float32)
    d = ((-z2_ref[...]) - e2_ref[...]) + 2.0 * dot
    half = NUM_CODES // 2
    d_lo = d[:, :half]
    d_hi = d[:, half:]
    col = lax.broadcasted_iota(jnp.int32, d_lo.shape, 1)
    m_lo = jnp.max(d_lo, axis=1, keepdims=True)
    i_lo = jnp.min(jnp.where(d_lo == m_lo, col, half), axis=1, keepdims=True)
    m_hi = jnp.max(d_hi, axis=1, keepdims=True)
    i_hi = half + jnp.min(jnp.where(d_hi == m_hi, col, half),
                          axis=1, keepdims=True)
    b1 = m_lo.astype(jnp.bfloat16).astype(jnp.float32)
    idx = jnp.where(m_hi > b1, i_hi, i_lo)
    idx_ref[...] = idx
    full_col = lax.broadcasted_iota(jnp.int32, d.shape, 1)
    enc_ref[...] = (full_col == idx).astype(jnp.float32)


def _vq_argmax_onehot(zf, z2, et, interpret=False):
    return pl.pallas_call(
        _vq_block,
        grid=(NUM_ROWS // ROW_BLK,),
        in_specs=[
            pl.BlockSpec((ROW_BLK, DIM), lambda i: (i, 0)),
            pl.BlockSpec((ROW_BLK, 1), lambda i: (i, 0)),
            pl.BlockSpec((DIM, NUM_CODES), lambda i: (0, 0)),
        ],
        out_specs=[
            pl.BlockSpec((ROW_BLK, NUM_CODES), lambda i: (i, 0)),
            pl.BlockSpec((ROW_BLK, 1), lambda i: (i, 0)),
        ],
        out_shape=[
            jax.ShapeDtypeStruct((NUM_ROWS, NUM_CODES), jnp.float32),
            jax.ShapeDtypeStruct((NUM_ROWS, 1), jnp.int32),
        ],
        scratch_shapes=[pltpu.VMEM((1, NUM_CODES), jnp.float32)],
        interpret=interpret,
    )(zf, z2, et)


def _make_sc_gather():
    info = plsc.get_sparse_core_info()
    nw = info.num_cores * info.num_subcores
    rows_per_w = NUM_ROWS // nw
    mesh = plsc.VectorSubcoreMesh(core_axis_name="c", subcore_axis_name="s")

    @functools.partial(
        pl.kernel,
        mesh=mesh,
        compiler_params=pltpu.CompilerParams(use_tc_tiling_on_sc=False),
        out_type=jax.ShapeDtypeStruct((NUM_ROWS, DIM), jnp.float32),
        scratch_types=[
            pltpu.VMEM((rows_per_w,), jnp.int32),
            pltpu.VMEM((rows_per_w, DIM), jnp.float32),
            pltpu.SemaphoreType.DMA,
        ],
    )
    def _gather_rows(table_hbm, idx_hbm, out_hbm, idx_v, rows_v, sem):
        wid = lax.axis_index("s") * info.num_cores + lax.axis_index("c")
        base = wid * rows_per_w
        pltpu.sync_copy(idx_hbm.at[pl.ds(base, rows_per_w)], idx_v)
        pltpu.async_copy(table_hbm.at[idx_v], rows_v, sem).wait()
        pltpu.sync_copy(rows_v, out_hbm.at[pl.ds(base, rows_per_w)])

    return _gather_rows


def kernel(z, embedding_weight):
    b, c, h, w = z.shape
    zf = jnp.transpose(z, (0, 2, 3, 1)).reshape(-1, c)
    z2 = jnp.sum(zf ** 2, axis=1, keepdims=True)
    et = embedding_weight.T
    enc, idx = _vq_argmax_onehot(zf, z2, et)
    idx_flat = idx.reshape(-1)
    zq_flat = _make_sc_gather()(embedding_weight, idx_flat)
    z_q = jnp.transpose(zq_flat.reshape(b, h, w, c), (0, 3, 1, 2))
    loss = jnp.array(0.0, dtype=z.dtype)
    return (z_q, loss, enc, idx_flat.reshape(b, h, w))
